# baseline (speedup 1.0000x reference)
import functools

import numpy as np

import jax
import jax.numpy as jnp
from jax import lax
from jax.experimental import pallas as pl
from jax.experimental.pallas import tpu as pltpu
from jax.experimental.pallas import tpu_sc as plsc

NC = 2
NS = 16
LANES = 16

_COLS_E = np.arange(128).reshape(4, 2, 16)[:, 0, :].reshape(64)
_COLS_O = np.arange(128).reshape(4, 2, 16)[:, 1, :].reshape(64)


def _pack_bf16(ue, uo):
    ie = lax.bitcast_convert_type(
        ue.astype(jnp.bfloat16).astype(jnp.float32), jnp.uint32)
    io = lax.bitcast_convert_type(
        uo.astype(jnp.bfloat16).astype(jnp.float32), jnp.uint32)
    return lax.bitcast_convert_type((ie >> 16) | io, jnp.int32)


def _encode_body(x_ref, w_ref, b_ref, o_ref):
    o_ref[...] = (
        jnp.dot(x_ref[...], w_ref[...], preferred_element_type=jnp.float32)
        + b_ref[...]
    )


def _encode(x, w, b):
    n, d = x.shape
    return pl.pallas_call(
        _encode_body,
        out_shape=jax.ShapeDtypeStruct((n, d), jnp.float32),
    )(x, w, b)


def _edge_emb_body(a_ref, we_ref, be_ref, wo_ref, bo_ref, o_ref):
    a = a_ref[...]
    ue = jnp.dot(a, we_ref[...], preferred_element_type=jnp.float32) + be_ref[...]
    uo = jnp.dot(a, wo_ref[...], preferred_element_type=jnp.float32) + bo_ref[...]
    o_ref[...] = _pack_bf16(ue, uo)


def _edge_emb(edge_attr, we, be, wo, bo, block_e):
    de, dh = we.shape
    e = edge_attr.shape[0]
    return pl.pallas_call(
        _edge_emb_body,
        grid=(e // block_e,),
        in_specs=[
            pl.BlockSpec((block_e, de), lambda i: (i, 0)),
            pl.BlockSpec((de, dh), lambda i: (0, 0)),
            pl.BlockSpec((1, dh), lambda i: (0, 0)),
            pl.BlockSpec((de, dh), lambda i: (0, 0)),
            pl.BlockSpec((1, dh), lambda i: (0, 0)),
        ],
        out_specs=pl.BlockSpec((block_e, dh), lambda i: (i, 0)),
        out_shape=jax.ShapeDtypeStruct((e, dh), jnp.int32),
    )(edge_attr, we, be.reshape(1, dh), wo, bo.reshape(1, dh))


def _layer_body(h_ref, agg_ref, w1_ref, b1_ref, w2_ref, b2_ref, g_ref,
                bt_ref, eps_ref, o_ref):
    h = h_ref[...]
    agg = agg_ref[0] + agg_ref[1]
    z = h * (1.0 + eps_ref[0, 0]) + agg
    t = jnp.maximum(
        jnp.dot(z, w1_ref[...], preferred_element_type=jnp.float32)
        + b1_ref[...], 0.0)
    u = jnp.dot(t, w2_ref[...], preferred_element_type=jnp.float32) + b2_ref[...]
    mean = jnp.mean(u, axis=0, keepdims=True)
    var = jnp.mean((u - mean) * (u - mean), axis=0, keepdims=True)
    o_ref[...] = (u - mean) * lax.rsqrt(var + 1e-5) * g_ref[...] + bt_ref[...] + h


def _layer(h, agg, w1, b1, w2, b2, gamma, beta, eps_l):
    n, d = h.shape
    return pl.pallas_call(
        _layer_body,
        out_shape=jax.ShapeDtypeStruct((n, d), jnp.float32),
    )(h, agg, w1, b1, w2, b2, gamma, beta, eps_l)


def _sc_message_pass(h, embb, src, dst, n, d, chunk=80):
    dh = d // 2
    e = src.shape[0]
    nw = NC * NS
    ew = e // nw
    nchunk = ew // chunk
    assert ew % chunk == 0 and chunk % 8 == 0 and chunk <= 128
    nrow = 3
    nemb = 2
    nidx = 4
    npt = (n // NS) // 8 * 8
    tail = n - npt * NS
    zrows = 48
    assert npt % zrows == 0 and tail % 8 == 0 and tail <= zrows
    pairs = d // (2 * LANES)

    mesh = plsc.VectorSubcoreMesh(core_axis_name="c", subcore_axis_name="s")

    @functools.partial(
        pl.kernel,
        out_type=jax.ShapeDtypeStruct((NC, n, d), jnp.float32),
        mesh=mesh,
        compiler_params=pltpu.CompilerParams(use_tc_tiling_on_sc=False),
        scratch_types=[
            [pltpu.VMEM((chunk,), jnp.int32)] * nidx,
            [pltpu.VMEM((chunk,), jnp.int32)] * nidx,
            [pltpu.VMEM((chunk, d), jnp.float32)] * nrow,
            [pltpu.VMEM((chunk, dh), jnp.int32)] * nemb,
            pltpu.VMEM((zrows, d), jnp.float32),
            pltpu.VMEM_SHARED((n, d), jnp.float32),
            [pltpu.SemaphoreType.DMA] * nidx,
            [pltpu.SemaphoreType.DMA] * nrow,
            [pltpu.SemaphoreType.DMA] * nemb,
            [pltpu.SemaphoreType.DMA] * nrow,
        ],
    )
    def body(h_hbm, emb_hbm, src_hbm, dst_hbm, out_hbm,
             srcv, dstv, rows, embv, zbuf, aggs, isem, gsem, esem, ssem):
        cid = lax.axis_index("c")
        sid = lax.axis_index("s")
        wid = sid * NC + cid

        @plsc.parallel_loop(0, zrows, 1, unroll=4)
        def _(i):
            for j in range(d // LANES):
                zbuf[i, pl.ds(j * LANES, LANES)] = jnp.zeros(
                    (LANES,), jnp.float32)
        for k in range(npt // zrows):
            pltpu.sync_copy(zbuf, aggs.at[pl.ds(sid * npt + k * zrows, zrows)])
        if tail:
            @pl.when(sid == 0)
            def _():
                pltpu.sync_copy(zbuf.at[pl.ds(0, tail)],
                                aggs.at[pl.ds(NS * npt, tail)])
        plsc.subcore_barrier()

        def idx_loads(c, i4):
            base = wid * ew + c * chunk
            pltpu.async_copy(src_hbm.at[pl.ds(base, chunk)], srcv[i4],
                             isem[i4])
            pltpu.async_copy(dst_hbm.at[pl.ds(base, chunk)], dstv[i4],
                             isem[i4])

        def data_loads(c, r, e2, i4):
            base = wid * ew + c * chunk
            pltpu.make_async_copy(src_hbm.at[pl.ds(base, chunk)], srcv[i4],
                                  isem[i4]).wait()
            pltpu.make_async_copy(dst_hbm.at[pl.ds(base, chunk)], dstv[i4],
                                  isem[i4]).wait()
            pltpu.async_copy(h_hbm.at[srcv[i4]], rows[r], gsem[r])
            pltpu.async_copy(emb_hbm.at[pl.ds(base, chunk), :], embv[e2],
                             esem[e2])

        def wait_scatter(r, i4):
            pltpu.make_async_copy(rows[r], aggs.at[dstv[i4]], ssem[r]).wait()

        def process(c, r, e2, i4):
            base = wid * ew + c * chunk
            pltpu.make_async_copy(h_hbm.at[srcv[i4]], rows[r], gsem[r]).wait()
            pltpu.make_async_copy(emb_hbm.at[pl.ds(base, chunk), :], embv[e2],
                                  esem[e2]).wait()

            @plsc.parallel_loop(0, chunk, 1, unroll=4)
            def _(i):
                for p in range(pairs):
                    ev = embv[e2][i, pl.ds(LANES * p, LANES)]
                    e0 = lax.bitcast_convert_type(ev << 16, jnp.float32)
                    e1 = lax.bitcast_convert_type(
                        ev & jnp.int32(-65536), jnp.float32)
                    s0 = pl.ds(32 * p, LANES)
                    s1 = pl.ds(32 * p + LANES, LANES)
                    rows[r][i, s0] = jnp.maximum(rows[r][i, s0] + e0, 0.0)
                    rows[r][i, s1] = jnp.maximum(rows[r][i, s1] + e1, 0.0)
            pltpu.async_copy(rows[r], aggs.at[dstv[i4]], ssem[r], add=True)

        idx_loads(0, 0)
        idx_loads(1, 1)
        data_loads(0, 0, 0, 0)

        ncyc = 12
        nmain = (nchunk - 2) // ncyc

        def round_body(g, carry):
            for k in range(ncyc):
                c = g * ncyc + k
                process(c, k % nrow, k % nemb, k % nidx)

                @pl.when(c >= 2)
                def _():
                    wait_scatter((k + 1) % nrow, (k + 2) % nidx)
                data_loads(c + 1, (k + 1) % nrow, (k + 1) % nemb,
                           (k + 1) % nidx)
                idx_loads(c + 2, (k + 2) % nidx)
            return carry
        lax.fori_loop(0, nmain, round_body, 0)

        for c in range(nmain * ncyc, nchunk):
            process(c, c % nrow, c % nemb, c % nidx)
            if c >= 2:
                wait_scatter((c - 2) % nrow, (c - 2) % nidx)
            if c + 1 < nchunk:
                data_loads(c + 1, (c + 1) % nrow, (c + 1) % nemb,
                           (c + 1) % nidx)
            if c + 2 < nchunk:
                idx_loads(c + 2, (c + 2) % nidx)
        wait_scatter((nchunk - 2) % nrow, (nchunk - 2) % nidx)
        wait_scatter((nchunk - 1) % nrow, (nchunk - 1) % nidx)

        plsc.subcore_barrier()
        for k in range(npt // zrows):
            r0 = sid * npt + k * zrows
            pltpu.sync_copy(aggs.at[pl.ds(r0, zrows)],
                            out_hbm.at[cid, pl.ds(r0, zrows), :])
        if tail:
            @pl.when(sid == 0)
            def _():
                pltpu.sync_copy(aggs.at[pl.ds(NS * npt, tail)],
                                out_hbm.at[cid, pl.ds(NS * npt, tail), :])

    return body(h, embb, src, dst)


def kernel(x, edge_index, edge_attr, W_init, b_init, W_edge, b_edge, eps,
           W1, b1, W2, b2, gamma, beta):
    num_l = W_edge.shape[0]
    n, d = x.shape
    src = edge_index[0]
    dst = edge_index[1]
    ce = jnp.asarray(_COLS_E)
    co = jnp.asarray(_COLS_O)

    h = _encode(x, W_init, b_init)

    for l in range(num_l):
        emb_l = _edge_emb(edge_attr, W_edge[l][:, ce], b_edge[l][ce],
                          W_edge[l][:, co], b_edge[l][co], block_e=8000)
        agg = _sc_message_pass(h, emb_l, src, dst, n, d)
        eps_l = eps[l].reshape(1, 1)
        h = _layer(h, agg, W1[l], b1[l], W2[l], b2[l],
                   gamma[l].reshape(1, -1), beta[l].reshape(1, -1), eps_l)
    return h
